# baseline (speedup 1.0000x reference)
import functools

import jax
import jax.numpy as jnp
from jax import lax
from jax.experimental import pallas as pl
from jax.experimental.pallas import tpu as pltpu
from jax.experimental.pallas import tpu_sc as plsc

OUT_DIM = 32
BATCH = 16384
FIELDS = 26
IN_DIM = 1000000

NC, NS = 2, 16
NW = NC * NS
L = 16
NBUF = 4

TCOLS = IN_DIM // 128
TCOL_REM = IN_DIM - TCOLS * 128
SUPER = 1
COLS_PER_W = TCOLS // NW
SUPERS_PER_W = COLS_PER_W // SUPER
K1_QUADS = SUPERS_PER_W // NBUF
K1_TAIL = SUPERS_PER_W - K1_QUADS * NBUF
EXTRA_COLS = TCOLS - COLS_PER_W * NW
SW = 128 * SUPER

NSTRIPE = FIELDS * (BATCH // 128)
STRIPES_PER_W = NSTRIPE // NW
K2_QUADS = STRIPES_PER_W // NBUF
IDX_PER_W = STRIPES_PER_W * 128

_mesh = plsc.VectorSubcoreMesh(core_axis_name="c", subcore_axis_name="s")
_params = pltpu.CompilerParams(use_tc_tiling_on_sc=True, needs_layout_passes=False)


def _wid():
    return lax.axis_index("s") * NC + lax.axis_index("c")


@functools.partial(
    pl.kernel,
    mesh=_mesh,
    out_type=jax.ShapeDtypeStruct((IN_DIM // 4, 128), jnp.float32),
    scratch_types=[
        [pltpu.VMEM((OUT_DIM, SW), jnp.float32)] * NBUF,
        [pltpu.VMEM((SW // 4, 128), jnp.float32)] * NBUF,
        [pltpu.SemaphoreType.DMA] * NBUF,
        [pltpu.SemaphoreType.DMA] * NBUF,
    ],
    compiler_params=_params,
)
def _relayout_kernel(tableT_hbm, tail4_hbm, tmp4_hbm, tileb, rowsb, isems, osems):
    wid = _wid()
    base = wid * COLS_PER_W

    def in_slice(sb):
        return tableT_hbm.at[:, pl.ds(base * 128 + sb * SW, SW)]

    def out_slice(sb):
        return tmp4_hbm.at[pl.ds(base * 32 + sb * (SW // 4), SW // 4)]

    def transpose_block(b, nrows):
        for g in range(nrows // L):
            rr16 = lax.iota(jnp.int32, L) + g * L
            i16 = rr16 >> 2
            j16 = (rr16 & 3) * OUT_DIM
            for d in range(OUT_DIM):
                vals = tileb[b][d, pl.ds(g * L, L)]
                plsc.store_scatter(rowsb[b], [i16, j16 + d], vals)

    def step(t, b):
        @pl.when(t + 2 < SUPERS_PER_W)
        def _():
            pltpu.async_copy(in_slice(t + 2), tileb[(b + 2) % NBUF],
                             isems[(b + 2) % NBUF])

        pltpu.make_async_copy(in_slice(t), tileb[b], isems[b]).wait()

        @pl.when(t >= NBUF)
        def _():
            pltpu.make_async_copy(rowsb[b], out_slice(t - NBUF), osems[b]).wait()

        transpose_block(b, SW)
        pltpu.async_copy(rowsb[b], out_slice(t), osems[b])

    pltpu.async_copy(in_slice(0), tileb[0], isems[0])
    pltpu.async_copy(in_slice(1), tileb[1], isems[1])

    def quad(q, carry):
        for u in range(NBUF):
            step(q * NBUF + u, u)
        return carry

    lax.fori_loop(0, K1_QUADS, quad, 0)
    for u in range(K1_TAIL):
        step(K1_QUADS * NBUF + u, u)
    for t in range(SUPERS_PER_W - NBUF, SUPERS_PER_W):
        pltpu.make_async_copy(rowsb[t % NBUF], out_slice(t), osems[t % NBUF]).wait()

    for w in range(EXTRA_COLS):
        @pl.when(wid == w)
        def _():
            col = COLS_PER_W * NW + w
            pltpu.sync_copy(
                tableT_hbm.at[:, pl.ds(col * 128, 128)],
                tileb[0].at[:, pl.ds(0, 128)],
            )
            transpose_block(0, 128)
            pltpu.sync_copy(
                rowsb[0].at[pl.ds(0, 32)],
                tmp4_hbm.at[pl.ds(col * 32, 32)],
            )

    @pl.when(wid == EXTRA_COLS)
    def _():
        pltpu.sync_copy(tail4_hbm, rowsb[0].at[pl.ds(0, TCOL_REM // 4)])
        pltpu.sync_copy(
            rowsb[0].at[pl.ds(0, TCOL_REM // 4)],
            tmp4_hbm.at[pl.ds(TCOLS * 32, TCOL_REM // 4)],
        )


@functools.partial(
    pl.kernel,
    mesh=_mesh,
    out_type=jax.ShapeDtypeStruct((FIELDS, OUT_DIM, BATCH), jnp.float32),
    scratch_types=[
        pltpu.VMEM((IDX_PER_W,), jnp.int32),
        pltpu.VMEM((IDX_PER_W,), jnp.int32),
        [pltpu.VMEM((128, 128), jnp.float32)] * NBUF,
        [pltpu.VMEM((OUT_DIM, 128), jnp.float32)] * NBUF,
        [pltpu.SemaphoreType.DMA] * NBUF,
        [pltpu.SemaphoreType.DMA] * NBUF,
    ],
    compiler_params=_params,
)
def _gather_kernel(tmp4_hbm, idxflat_hbm, out3_hbm, idxv, idx4v, blkb, stripeb,
                   gsems, ssems):
    wid = _wid()
    s0 = wid * STRIPES_PER_W

    pltpu.sync_copy(idxflat_hbm.at[pl.ds(s0 * 128, IDX_PER_W)], idxv)

    def mk4(g, carry):
        idx4v[pl.ds(g * L, L)] = idxv[pl.ds(g * L, L)] >> 2
        return carry

    lax.fori_loop(0, IDX_PER_W // L, mk4, 0)

    def gather_start(t, b):
        pltpu.async_copy(
            tmp4_hbm.at[idx4v.at[pl.ds(t * 128, 128)]], blkb[b], gsems[b]
        )

    def transmute(t, b):
        for g in range(128 // L):
            b16 = lax.iota(jnp.int32, L) + g * L
            col16 = (idxv[pl.ds(t * 128 + g * L, L)] & 3) * OUT_DIM
            for d in range(OUT_DIM):
                vals = plsc.load_gather(blkb[b], [b16, col16 + d])
                stripeb[b][d, pl.ds(g * L, L)] = vals

    def out_slice(s):
        f = s >> 7
        k = s & 127
        return out3_hbm.at[f, :, pl.ds(k * 128, 128)]

    def step(t, b):
        @pl.when(t + 2 < STRIPES_PER_W)
        def _():
            gather_start(t + 2, (b + 2) % NBUF)

        pltpu.make_async_copy(
            tmp4_hbm.at[idx4v.at[pl.ds(t * 128, 128)]], blkb[b], gsems[b]
        ).wait()

        @pl.when(t >= NBUF)
        def _():
            pltpu.make_async_copy(
                stripeb[b], out_slice(s0 + t - NBUF), ssems[b]
            ).wait()

        transmute(t, b)
        pltpu.async_copy(stripeb[b], out_slice(s0 + t), ssems[b])

    gather_start(0, 0)
    gather_start(1, 1)

    def quad(q, carry):
        for u in range(NBUF):
            step(q * NBUF + u, u)
        return carry

    lax.fori_loop(0, K2_QUADS, quad, 0)
    for t in range(STRIPES_PER_W - NBUF, STRIPES_PER_W):
        pltpu.make_async_copy(
            stripeb[t % NBUF], out_slice(s0 + t), ssems[t % NBUF]
        ).wait()


def kernel(in_tensor, table):
    tableT = table.T
    idxT = in_tensor.T.astype(jnp.int32)
    idxflat = idxT.reshape(-1)
    tail4 = table[TCOLS * 128:].reshape(TCOL_REM // 4, 128)
    tmp4 = _relayout_kernel(tableT, tail4)
    out3 = _gather_kernel(tmp4, idxflat)
    return out3.transpose(2, 0, 1)


# speedup vs baseline: 1.2690x; 1.2690x over previous
import functools

import jax
import jax.numpy as jnp
from jax import lax
from jax.experimental import pallas as pl
from jax.experimental.pallas import tpu as pltpu
from jax.experimental.pallas import tpu_sc as plsc

OUT_DIM = 32
BATCH = 16384
FIELDS = 26

B = BATCH * FIELDS
NC, NS = 2, 16
NW = NC * NS
B_PER_W = B // NW
CHUNK = 832
N_CHUNKS = B_PER_W // CHUNK
NBUF = 4

_mesh = plsc.VectorSubcoreMesh(core_axis_name="c", subcore_axis_name="s")


@functools.partial(
    pl.kernel,
    mesh=_mesh,
    out_type=jax.ShapeDtypeStruct((B, OUT_DIM), jnp.float32),
    scratch_types=[
        pltpu.VMEM((B_PER_W,), jnp.int32),
        pltpu.VMEM((NBUF, CHUNK, OUT_DIM), jnp.float32),
        [pltpu.SemaphoreType.DMA] * NBUF,
        [pltpu.SemaphoreType.DMA] * NBUF,
    ],
    compiler_params=pltpu.CompilerParams(use_tc_tiling_on_sc=False),
)
def _gather_kernel(idx_hbm, table_hbm, out_hbm, idx_v, rows_v, gsems, ssems):
    wid = lax.axis_index("s") * NC + lax.axis_index("c")
    base = wid * B_PER_W

    pltpu.sync_copy(idx_hbm.at[pl.ds(base, B_PER_W)], idx_v)

    gathers = [None] * N_CHUNKS
    stores = [None] * N_CHUNKS

    def start_gather(i):
        b = i % NBUF
        gathers[i] = pltpu.async_copy(
            table_hbm.at[idx_v.at[pl.ds(i * CHUNK, CHUNK)]],
            rows_v.at[b],
            gsems[b],
        )

    def start_store(i):
        b = i % NBUF
        stores[i] = pltpu.async_copy(
            rows_v.at[b],
            out_hbm.at[pl.ds(base + i * CHUNK, CHUNK)],
            ssems[b],
        )

    for i in range(NBUF - 1):
        start_gather(i)
    for i in range(N_CHUNKS):
        nxt = i + NBUF - 1
        if nxt < N_CHUNKS:
            if i >= 1:
                stores[i - 1].wait()
                stores[i - 1] = None
            start_gather(nxt)
        gathers[i].wait()
        start_store(i)
    for s in stores:
        if s is not None:
            s.wait()


def kernel(in_tensor, table):
    idx = in_tensor.reshape(-1).astype(jnp.int32)
    out = _gather_kernel(idx, table)
    return out.reshape(BATCH, FIELDS, OUT_DIM)
